# baseline (speedup 1.0000x reference)
import functools

import jax
import jax.numpy as jnp
from jax import lax
from jax.experimental import pallas as pl
from jax.experimental.pallas import tpu as pltpu
from jax.experimental.pallas import tpu_sc as plsc

_NUM_E = 1000000
_H = 64
_BATCH = 16384
_NW = 32
_BPW = _BATCH // _NW
_CHUNK = 128
_NCHUNK = _BPW // _CHUNK


def _gate_body(table_hbm, idx_hbm, w_hbm, b_hbm, out_hbm,
               idx_v, rows_v, w_v, b_v, t_v, out_v, sem):
    wid = lax.axis_index("s") * 2 + lax.axis_index("c")
    base = wid * _BPW

    pltpu.sync_copy(idx_hbm.at[wid], idx_v)
    pltpu.sync_copy(w_hbm, w_v)
    pltpu.sync_copy(b_hbm, b_v)

    copies = [
        pltpu.async_copy(
            table_hbm.at[idx_v.at[j]],
            rows_v.at[pl.ds(j * _CHUNK, _CHUNK)],
            sem,
        )
        for j in range(_NCHUNK)
    ]
    for c in copies:
        c.wait()

    wv = [w_v[pl.ds(16 * c, 16)] for c in range(4)]
    bv = b_v[...]
    lanes = lax.iota(jnp.int32, 16)
    zv = jnp.zeros((16,), jnp.float32)

    for r in range(16):
        t_v[pl.ds(r * 48, 16)] = zv
        t_v[pl.ds(r * 48 + 32, 16)] = zv

    def body(g, carry):
        acc = zv
        for r in range(16):
            i = g * 16 + r
            p = rows_v[i, pl.ds(0, 16)] * wv[0]
            p += rows_v[i, pl.ds(16, 16)] * wv[1]
            p += rows_v[i, pl.ds(32, 16)] * wv[2]
            p += rows_v[i, pl.ds(48, 16)] * wv[3]
            s = p
            mid = r * 48 + 16
            for d in (8, 4, 2, 1):
                t_v[pl.ds(mid, 16)] = s
                off = -d if (r & d) else d
                s = s + t_v[pl.ds(mid + off, 16)]
            acc = jnp.where(lanes == r, s, acc)
        x = acc + bv
        out_v[pl.ds(g * 16, 16)] = 1.0 / (1.0 + jnp.exp(-x))
        return carry

    lax.fori_loop(0, _BPW // 16, body, 0)

    pltpu.sync_copy(out_v, out_hbm.at[pl.ds(base, _BPW)])


@jax.jit
def _gate_sc(idx, table, w_flat, b_vec):
    mesh = plsc.VectorSubcoreMesh(core_axis_name="c", subcore_axis_name="s")
    f = functools.partial(
        pl.kernel,
        mesh=mesh,
        compiler_params=pltpu.CompilerParams(use_tc_tiling_on_sc=False),
        out_type=jax.ShapeDtypeStruct((_BATCH,), jnp.float32),
        scratch_types=[
            pltpu.VMEM((_NCHUNK, _CHUNK), jnp.int32),
            pltpu.VMEM((_BPW, _H), jnp.float32),
            pltpu.VMEM((_H,), jnp.float32),
            pltpu.VMEM((16,), jnp.float32),
            pltpu.VMEM((16 * 48,), jnp.float32),
            pltpu.VMEM((_BPW,), jnp.float32),
            pltpu.SemaphoreType.DMA,
        ],
    )(_gate_body)
    return f(table, idx, w_flat, b_vec)


def kernel(X, Y, gate_theta, W, b):
    idx = X.reshape(_NW, _NCHUNK, _CHUNK)
    w_flat = W.reshape(_H)
    b_vec = jnp.broadcast_to(b.reshape(()), (16,)).astype(jnp.float32)
    out = _gate_sc(idx, gate_theta, w_flat, b_vec)
    return out.reshape(_BATCH, 1)
